# baseline (speedup 1.0000x reference)
import functools

import jax
import jax.numpy as jnp
from jax import lax
from jax.experimental import pallas as pl
from jax.experimental.pallas import tpu as pltpu
from jax.experimental.pallas import tpu_sc as plsc

D = 768
SEQ = 20
NKEY = 4 * 2 * SEQ
NC, NS = 2, 16
NW = NC * NS
CHUNK = 128


def _lut_body(x_ref, seg_ref, tok_ref, pos_ref, seg_t_ref, gamma_ref, beta_ref,
              lut_ref, idx_ref):
    l_iota = lax.broadcasted_iota(jnp.int32, x_ref.shape, 1)
    idx_ref[...] = x_ref[...] * (2 * SEQ) + seg_ref[...] * SEQ + l_iota

    r = lax.broadcasted_iota(jnp.int32, (NKEY, 1), 0)
    oh_v = (r // (2 * SEQ) == lax.broadcasted_iota(jnp.int32, (NKEY, 4), 1)).astype(jnp.float32)
    oh_s = ((r % (2 * SEQ)) // SEQ == lax.broadcasted_iota(jnp.int32, (NKEY, 2), 1)).astype(jnp.float32)
    oh_l = (r % SEQ == lax.broadcasted_iota(jnp.int32, (NKEY, SEQ), 1)).astype(jnp.float32)
    emb = (jnp.dot(oh_v, tok_ref[...], preferred_element_type=jnp.float32)
           + jnp.dot(oh_s, seg_t_ref[...], preferred_element_type=jnp.float32)
           + jnp.dot(oh_l, pos_ref[0:SEQ, :], preferred_element_type=jnp.float32))
    mean = jnp.mean(emb, axis=-1, keepdims=True)
    c = emb - mean
    var = jnp.mean(c * c, axis=-1, keepdims=True)
    inv = lax.rsqrt(var + 1e-5)
    lut_ref[...] = c * inv * gamma_ref[0, :][None, :] + beta_ref[0, :][None, :]


def _build_lut_and_idx(x, seg, tok_table, pos_table, seg_table, gamma, beta):
    b, seq_len = x.shape
    d = tok_table.shape[1]
    return pl.pallas_call(
        _lut_body,
        in_specs=[
            pl.BlockSpec(x.shape, lambda: (0, 0)),
            pl.BlockSpec(seg.shape, lambda: (0, 0)),
            pl.BlockSpec(tok_table.shape, lambda: (0, 0)),
            pl.BlockSpec(pos_table.shape, lambda: (0, 0)),
            pl.BlockSpec(seg_table.shape, lambda: (0, 0)),
            pl.BlockSpec((1, d), lambda: (0, 0)),
            pl.BlockSpec((1, d), lambda: (0, 0)),
        ],
        out_specs=[
            pl.BlockSpec((NKEY, d), lambda: (0, 0)),
            pl.BlockSpec((b, seq_len), lambda: (0, 0)),
        ],
        out_shape=[
            jax.ShapeDtypeStruct((NKEY, d), jnp.float32),
            jax.ShapeDtypeStruct((b, seq_len), jnp.int32),
        ],
    )(x, seg, tok_table, pos_table, seg_table,
      gamma.reshape(1, d), beta.reshape(1, d))


def _sc_gather(lut, idx_flat):
    n = idx_flat.shape[0]
    rows_per_w = n // NW
    n_chunks = rows_per_w // CHUNK
    mesh = plsc.VectorSubcoreMesh(core_axis_name="c", subcore_axis_name="s",
                                  num_cores=NC, num_subcores=NS)

    @functools.partial(
        pl.kernel,
        mesh=mesh,
        out_type=jax.ShapeDtypeStruct((n, D), jnp.float32),
        scratch_types=[
            pltpu.VMEM((CHUNK,), jnp.int32),
            pltpu.VMEM((CHUNK, D), jnp.float32),
            pltpu.SemaphoreType.DMA,
        ],
    )
    def k(lut_hbm, idx_hbm, out_hbm, idx_v, rows_v, sem):
        wid = lax.axis_index("s") * NC + lax.axis_index("c")
        base = wid * rows_per_w

        def body(g, carry):
            off = base + g * CHUNK
            pltpu.sync_copy(idx_hbm.at[pl.ds(off, CHUNK)], idx_v)
            pltpu.async_copy(lut_hbm.at[idx_v], rows_v, sem).wait()
            pltpu.sync_copy(rows_v, out_hbm.at[pl.ds(off, CHUNK)])
            return carry

        lax.fori_loop(0, n_chunks, body, 0)

    return k(lut, idx_flat)


def kernel(x, seg, tok_table, pos_table, seg_table, gamma, beta):
    b, seq_len = x.shape
    d = tok_table.shape[1]
    lut, idx = _build_lut_and_idx(x, seg, tok_table, pos_table, seg_table,
                                  gamma, beta)
    out = _sc_gather(lut, idx.reshape(-1))
    return out.reshape(b, seq_len, d)


# speedup vs baseline: 1.0015x; 1.0015x over previous
import functools

import jax
import jax.numpy as jnp
from jax import lax
from jax.experimental import pallas as pl
from jax.experimental.pallas import tpu as pltpu
from jax.experimental.pallas import tpu_sc as plsc

D = 768
SEQ = 20
NKEY = 4 * 2 * SEQ
NC, NS = 2, 16
NW = NC * NS
CHUNK = 64


def _lut_body(x_ref, seg_ref, tok_ref, pos_ref, seg_t_ref, gamma_ref, beta_ref,
              lut_ref, idx_ref):
    l_iota = lax.broadcasted_iota(jnp.int32, x_ref.shape, 1)
    idx_ref[...] = x_ref[...] * (2 * SEQ) + seg_ref[...] * SEQ + l_iota

    r = lax.broadcasted_iota(jnp.int32, (NKEY, 1), 0)
    oh_v = (r // (2 * SEQ) == lax.broadcasted_iota(jnp.int32, (NKEY, 4), 1)).astype(jnp.float32)
    oh_s = ((r % (2 * SEQ)) // SEQ == lax.broadcasted_iota(jnp.int32, (NKEY, 2), 1)).astype(jnp.float32)
    oh_l = (r % SEQ == lax.broadcasted_iota(jnp.int32, (NKEY, SEQ), 1)).astype(jnp.float32)
    hi = lax.Precision.HIGHEST
    emb = (jnp.dot(oh_v, tok_ref[...], preferred_element_type=jnp.float32, precision=hi)
           + jnp.dot(oh_s, seg_t_ref[...], preferred_element_type=jnp.float32, precision=hi)
           + jnp.dot(oh_l, pos_ref[0:SEQ, :], preferred_element_type=jnp.float32, precision=hi))
    mean = jnp.mean(emb, axis=-1, keepdims=True)
    c = emb - mean
    var = jnp.mean(c * c, axis=-1, keepdims=True)
    inv = lax.rsqrt(var + 1e-5)
    lut_ref[...] = c * inv * gamma_ref[0, :][None, :] + beta_ref[0, :][None, :]


def _build_lut_and_idx(x, seg, tok_table, pos_table, seg_table, gamma, beta):
    b, seq_len = x.shape
    d = tok_table.shape[1]
    return pl.pallas_call(
        _lut_body,
        in_specs=[
            pl.BlockSpec(x.shape, lambda: (0, 0)),
            pl.BlockSpec(seg.shape, lambda: (0, 0)),
            pl.BlockSpec(tok_table.shape, lambda: (0, 0)),
            pl.BlockSpec(pos_table.shape, lambda: (0, 0)),
            pl.BlockSpec(seg_table.shape, lambda: (0, 0)),
            pl.BlockSpec((1, d), lambda: (0, 0)),
            pl.BlockSpec((1, d), lambda: (0, 0)),
        ],
        out_specs=[
            pl.BlockSpec((NKEY, d), lambda: (0, 0)),
            pl.BlockSpec((b, seq_len), lambda: (0, 0)),
        ],
        out_shape=[
            jax.ShapeDtypeStruct((NKEY, d), jnp.float32),
            jax.ShapeDtypeStruct((b, seq_len), jnp.int32),
        ],
    )(x, seg, tok_table, pos_table, seg_table,
      gamma.reshape(1, d), beta.reshape(1, d))


def _sc_gather(lut, idx_flat):
    n = idx_flat.shape[0]
    rows_per_w = n // NW
    n_chunks = rows_per_w // CHUNK
    mesh = plsc.VectorSubcoreMesh(core_axis_name="c", subcore_axis_name="s",
                                  num_cores=NC, num_subcores=NS)

    @functools.partial(
        pl.kernel,
        mesh=mesh,
        out_type=jax.ShapeDtypeStruct((n, D), jnp.float32),
        scratch_types=[
            pltpu.VMEM((rows_per_w,), jnp.int32),
            pltpu.VMEM((CHUNK, D), jnp.float32),
            pltpu.VMEM((CHUNK, D), jnp.float32),
            pltpu.SemaphoreType.DMA,
            pltpu.SemaphoreType.DMA,
            pltpu.SemaphoreType.DMA,
            pltpu.SemaphoreType.DMA,
        ],
    )
    def k(lut_hbm, idx_hbm, out_hbm, idx_v, rows0, rows1,
          sg0, sg1, ss0, ss1):
        sid = lax.axis_index("s")
        wid = sid * NC + lax.axis_index("c")
        base = wid * rows_per_w

        pltpu.sync_copy(idx_hbm.at[pl.ds(base, rows_per_w)], idx_v)

        rows = (rows0, rows1)
        sg = (sg0, sg1)
        ss = (ss0, ss1)

        def gather(g, p):
            return pltpu.async_copy(
                lut_hbm.at[idx_v.at[pl.ds(g * CHUNK, CHUNK)]], rows[p], sg[p])

        def store(g, p):
            return pltpu.async_copy(
                rows[p], out_hbm.at[pl.ds(base + g * CHUNK, CHUNK)], ss[p])

        def body(h, carry):
            for p in (0, 1):
                g = 2 * h + p

                @pl.when(h > 0)
                def _():
                    pltpu.make_async_copy(
                        rows[p], out_hbm.at[pl.ds(base + g * CHUNK, CHUNK)],
                        ss[p]).wait()

                gather(g, p).wait()
                store(g, p)
            return carry

        lax.fori_loop(0, n_chunks // 2, body, 0)
        for p in (0, 1):
            g = n_chunks - 2 + p
            pltpu.make_async_copy(
                rows[p], out_hbm.at[pl.ds(base + g * CHUNK, CHUNK)],
                ss[p]).wait()

    return k(lut, idx_flat)


def kernel(x, seg, tok_table, pos_table, seg_table, gamma, beta):
    b, seq_len = x.shape
    d = tok_table.shape[1]
    lut, idx = _build_lut_and_idx(x, seg, tok_table, pos_table, seg_table,
                                  gamma, beta)
    out = _sc_gather(lut, idx.reshape(-1))
    return out.reshape(b, seq_len, d)
